# initial kernel scaffold (unmeasured)
import jax
import jax.numpy as jnp
from jax import lax
from jax.experimental import pallas as pl
from jax.experimental.pallas import tpu as pltpu

H = 16
DH = 128
DR = 32


def kernel(x, Wdkv, Wuk, Wuv, Wq, Wqr, Wkr, Wo):
    B, S, D = x.shape
    Dc = Wdkv.shape[1]

    bf = jnp.bfloat16
    x2 = x.reshape(S, D).astype(bf)
    wdkv = Wdkv.astype(bf)
    wuk = Wuk.astype(bf)
    wuv = Wuv.astype(bf)
    wq = Wq.astype(bf)
    wqr = Wqr.astype(bf)
    wkr = Wkr.astype(bf)
    wo = Wo.astype(bf)

    def body(x_ref, wdkv_ref, wuk_ref, wuv_ref, wq_ref, wqr_ref, wkr_ref,
             wo_ref, out_ref, c_ref, c_recv, wuk_recv, wuv_recv,
             send_sems, recv_sems):
        my_x = lax.axis_index("x")
        my_y = lax.axis_index("y")
        my_z = lax.axis_index("z")
        peer = (my_x, my_y, 1 - my_z)

        barrier_sem = pltpu.get_barrier_semaphore()
        pl.semaphore_signal(barrier_sem, inc=1, device_id=peer,
                            device_id_type=pl.DeviceIdType.MESH)
        pl.semaphore_wait(barrier_sem, 1)

        xv = x_ref[...]
        c = jnp.dot(xv, wdkv_ref[...],
                    preferred_element_type=jnp.float32).astype(bf)
        c_ref[...] = c

        rdmas = []
        for i, (src, dst) in enumerate(
            [(c_ref, c_recv), (wuk_ref, wuk_recv), (wuv_ref, wuv_recv)]
        ):
            rdma = pltpu.make_async_remote_copy(
                src_ref=src, dst_ref=dst,
                send_sem=send_sems.at[i], recv_sem=recv_sems.at[i],
                device_id=peer, device_id_type=pl.DeviceIdType.MESH,
            )
            rdma.start()
            rdmas.append(rdma)

        q = jnp.dot(xv, wq_ref[...], preferred_element_type=jnp.float32
                    ).astype(bf)
        qr = jnp.dot(xv, wqr_ref[...], preferred_element_type=jnp.float32
                     ).astype(bf)
        kr = jnp.dot(xv, wkr_ref[...], preferred_element_type=jnp.float32
                     ).astype(bf)

        for rdma in rdmas:
            rdma.wait()

        k = (jnp.dot(c, wuk_ref[...], preferred_element_type=jnp.float32)
             + jnp.dot(c_recv[...], wuk_recv[...],
                       preferred_element_type=jnp.float32)).astype(bf)
        v = (jnp.dot(c, wuv_ref[...], preferred_element_type=jnp.float32)
             + jnp.dot(c_recv[...], wuv_recv[...],
                       preferred_element_type=jnp.float32)).astype(bf)

        scale = (DH + DR) ** -0.5
        krt = kr.T
        outs = []
        for h in range(H):
            qh = q[:, h * DH:(h + 1) * DH]
            kh = k[:, h * DH:(h + 1) * DH]
            qrh = qr[:, h * DR:(h + 1) * DR]
            s = (jnp.dot(qh, kh.T, preferred_element_type=jnp.float32)
                 + jnp.dot(qrh, krt, preferred_element_type=jnp.float32)
                 ) * scale
            m = jnp.max(s, axis=-1, keepdims=True)
            p = jnp.exp(s - m)
            p = (p / jnp.sum(p, axis=-1, keepdims=True)).astype(bf)
            vh = v[:, h * DH:(h + 1) * DH]
            outs.append(jnp.dot(p, vh, preferred_element_type=jnp.float32))
        o = jnp.concatenate(outs, axis=1).astype(bf)
        out_ref[...] = jnp.dot(o, wo_ref[...],
                               preferred_element_type=jnp.float32)

    out = pl.pallas_call(
        body,
        out_shape=jax.ShapeDtypeStruct((S, D), jnp.float32),
        in_specs=[pl.BlockSpec(memory_space=pltpu.VMEM)] * 8,
        out_specs=pl.BlockSpec(memory_space=pltpu.VMEM),
        scratch_shapes=[
            pltpu.VMEM((S, Dc), bf),
            pltpu.VMEM((S, Dc), bf),
            pltpu.VMEM((Dc, D), bf),
            pltpu.VMEM((Dc, D), bf),
            pltpu.SemaphoreType.DMA((3,)),
            pltpu.SemaphoreType.DMA((3,)),
        ],
        compiler_params=pltpu.CompilerParams(collective_id=0),
    )(x2, wdkv, wuk, wuv, wq, wqr, wkr, wo)

    return out.reshape(B, S, D)


# baseline (device time: 123777 ns/iter reference)
import jax
import jax.numpy as jnp
from jax import lax
from jax.experimental import pallas as pl
from jax.experimental.pallas import tpu as pltpu

H = 16
DH = 128
DR = 32


def kernel(x, Wdkv, Wuk, Wuv, Wq, Wqr, Wkr, Wo):
    B, S, D = x.shape
    Dc = Wdkv.shape[1]

    bf = jnp.bfloat16
    x2 = x.reshape(S, D).astype(bf)
    wdkv = Wdkv.astype(bf)
    wuk = Wuk.astype(bf)
    wuv = Wuv.astype(bf)
    wq = Wq.astype(bf)
    wqr = Wqr.astype(bf)
    wkr = Wkr.astype(bf)
    wo = Wo.astype(bf)

    def body(x_ref, wdkv_ref, wuk_ref, wuv_ref, wq_ref, wqr_ref, wkr_ref,
             wo_ref, out_ref, c_ref, c_recv, wuk_recv, wuv_recv,
             send_sems, recv_sems):
        my_x = lax.axis_index("x")
        my_y = lax.axis_index("y")
        my_z = lax.axis_index("z")
        peer = (my_x, my_y, 1 - my_z)

        barrier_sem = pltpu.get_barrier_semaphore()
        pl.semaphore_signal(barrier_sem, inc=1, device_id=peer,
                            device_id_type=pl.DeviceIdType.MESH)
        pl.semaphore_wait(barrier_sem, 1)

        xv = x_ref[...]
        c = jnp.dot(xv, wdkv_ref[...],
                    preferred_element_type=jnp.float32).astype(bf)
        c_ref[...] = c

        rdmas = []
        for i, (src, dst) in enumerate(
            [(c_ref, c_recv), (wuk_ref, wuk_recv), (wuv_ref, wuv_recv)]
        ):
            rdma = pltpu.make_async_remote_copy(
                src_ref=src, dst_ref=dst,
                send_sem=send_sems.at[i], recv_sem=recv_sems.at[i],
                device_id=peer, device_id_type=pl.DeviceIdType.MESH,
            )
            rdma.start()
            rdmas.append(rdma)

        q = jnp.dot(xv, wq_ref[...], preferred_element_type=jnp.float32
                    ).astype(bf)
        qr = jnp.dot(xv, wqr_ref[...], preferred_element_type=jnp.float32
                     ).astype(bf)
        kr = jnp.dot(xv, wkr_ref[...], preferred_element_type=jnp.float32
                     ).astype(bf)

        for rdma in rdmas:
            rdma.wait()

        k = (jnp.dot(c, wuk_ref[...], preferred_element_type=jnp.float32)
             + jnp.dot(c_recv[...], wuk_recv[...],
                       preferred_element_type=jnp.float32)).astype(bf)
        v = (jnp.dot(c, wuv_ref[...], preferred_element_type=jnp.float32)
             + jnp.dot(c_recv[...], wuv_recv[...],
                       preferred_element_type=jnp.float32)).astype(bf)

        scale = (DH + DR) ** -0.5
        krt = kr.T
        outs = []
        for h in range(H):
            qh = q[:, h * DH:(h + 1) * DH]
            kh = k[:, h * DH:(h + 1) * DH]
            qrh = qr[:, h * DR:(h + 1) * DR]
            s = (jnp.dot(qh, kh.T, preferred_element_type=jnp.float32)
                 + jnp.dot(qrh, krt, preferred_element_type=jnp.float32)
                 ) * scale
            m = jnp.max(s, axis=-1, keepdims=True)
            p = jnp.exp(s - m)
            p = (p / jnp.sum(p, axis=-1, keepdims=True)).astype(bf)
            vh = v[:, h * DH:(h + 1) * DH]
            outs.append(jnp.dot(p, vh, preferred_element_type=jnp.float32))
        o = jnp.concatenate(outs, axis=1).astype(bf)
        out_ref[...] = jnp.dot(o, wo_ref[...],
                               preferred_element_type=jnp.float32)

    out = pl.pallas_call(
        body,
        out_shape=jax.ShapeDtypeStruct((S, D), jnp.float32),
        in_specs=[pl.BlockSpec(memory_space=pltpu.VMEM)] * 8,
        out_specs=pl.BlockSpec(memory_space=pltpu.VMEM),
        scratch_shapes=[
            pltpu.VMEM((S, Dc), bf),
            pltpu.VMEM((S, Dc), bf),
            pltpu.VMEM((Dc, D), bf),
            pltpu.VMEM((Dc, D), bf),
            pltpu.SemaphoreType.DMA((3,)),
            pltpu.SemaphoreType.DMA((3,)),
        ],
        compiler_params=pltpu.CompilerParams(
            collective_id=0, vmem_limit_bytes=100 * 2**20
        ),
    )(x2, wdkv, wuk, wuv, wq, wqr, wkr, wo)

    return out.reshape(B, S, D)


# device time: 119087 ns/iter; 1.0394x vs baseline; 1.0394x over previous
import jax
import jax.numpy as jnp
from jax import lax
from jax.experimental import pallas as pl
from jax.experimental.pallas import tpu as pltpu

H = 16
DH = 128
DR = 32
NXY = 4


def kernel(x, Wdkv, Wuk, Wuv, Wq, Wqr, Wkr, Wo):
    B, S, D = x.shape
    Dc = Wdkv.shape[1]
    R = S // NXY

    bf = jnp.bfloat16
    x2 = x.reshape(S, D).astype(bf)
    wdkv = Wdkv.astype(bf)
    wuk = Wuk.astype(bf)
    wuv = Wuv.astype(bf)
    wq = Wq.astype(bf)
    wqr = Wqr.astype(bf)
    wkr = Wkr.astype(bf)
    wo = Wo.astype(bf)

    def body(x_ref, wdkv_ref, wuk_ref, wuv_ref, wq_ref, wqr_ref, wkr_ref,
             wo_ref, out_ref, c_ref, c_recv, wuk_recv, wuv_recv, oblk_ref,
             ra1, ra2, rb1, zs_sems, zr_sems, rs_sems, rr_sems):
        my_x = lax.axis_index("x")
        my_y = lax.axis_index("y")
        my_z = lax.axis_index("z")
        zpeer = (my_x, my_y, 1 - my_z)

        p = 2 * my_x + (my_x + my_y) % 2
        pr = (p + 1) % NXY
        plft = (p + 3) % NXY
        right = (pr // 2, (pr % 2 + pr // 2) % 2, my_z)
        left = (plft // 2, (plft % 2 + plft // 2) % 2, my_z)

        barrier_sem = pltpu.get_barrier_semaphore()
        for nbr in (zpeer, left, right):
            pl.semaphore_signal(barrier_sem, inc=1, device_id=nbr,
                                device_id_type=pl.DeviceIdType.MESH)
        pl.semaphore_wait(barrier_sem, 3)

        xv = x_ref[...]
        c = jnp.dot(xv, wdkv_ref[...],
                    preferred_element_type=jnp.float32).astype(bf)
        c_ref[...] = c

        zrdmas = []
        for i, (src, dst) in enumerate(
            [(c_ref, c_recv), (wuk_ref, wuk_recv), (wuv_ref, wuv_recv)]
        ):
            rdma = pltpu.make_async_remote_copy(
                src_ref=src, dst_ref=dst,
                send_sem=zs_sems.at[i], recv_sem=zr_sems.at[i],
                device_id=zpeer, device_id_type=pl.DeviceIdType.MESH,
            )
            rdma.start()
            zrdmas.append(rdma)

        xq = x_ref[pl.ds(p * R, R), :]
        q = jnp.dot(xq, wq_ref[...], preferred_element_type=jnp.float32
                    ).astype(bf)
        qr = jnp.dot(xq, wqr_ref[...], preferred_element_type=jnp.float32
                     ).astype(bf)
        kr = jnp.dot(xv, wkr_ref[...], preferred_element_type=jnp.float32
                     ).astype(bf)

        for rdma in zrdmas:
            rdma.wait()

        k = (jnp.dot(c, wuk_ref[...], preferred_element_type=jnp.float32)
             + jnp.dot(c_recv[...], wuk_recv[...],
                       preferred_element_type=jnp.float32)).astype(bf)
        v = (jnp.dot(c, wuv_ref[...], preferred_element_type=jnp.float32)
             + jnp.dot(c_recv[...], wuv_recv[...],
                       preferred_element_type=jnp.float32)).astype(bf)

        scale = (DH + DR) ** -0.5
        krt = kr.T
        outs = []
        for h in range(H):
            qh = q[:, h * DH:(h + 1) * DH]
            kh = k[:, h * DH:(h + 1) * DH]
            qrh = qr[:, h * DR:(h + 1) * DR]
            s = (jnp.dot(qh, kh.T, preferred_element_type=jnp.float32)
                 + jnp.dot(qrh, krt, preferred_element_type=jnp.float32)
                 ) * scale
            m = jnp.max(s, axis=-1, keepdims=True)
            pr_ = jnp.exp(s - m)
            pr_ = (pr_ / jnp.sum(pr_, axis=-1, keepdims=True)).astype(bf)
            vh = v[:, h * DH:(h + 1) * DH]
            outs.append(jnp.dot(pr_, vh, preferred_element_type=jnp.float32))
        o = jnp.concatenate(outs, axis=1).astype(bf)
        out_rows = jnp.dot(o, wo_ref[...],
                           preferred_element_type=jnp.float32)
        out_ref[pl.ds(p * R, R), :] = out_rows
        oblk_ref[...] = out_rows.astype(bf)

        a1 = pltpu.make_async_remote_copy(
            src_ref=oblk_ref, dst_ref=ra1,
            send_sem=rs_sems.at[0], recv_sem=rr_sems.at[0],
            device_id=right, device_id_type=pl.DeviceIdType.MESH,
        )
        b1 = pltpu.make_async_remote_copy(
            src_ref=oblk_ref, dst_ref=rb1,
            send_sem=rs_sems.at[1], recv_sem=rr_sems.at[1],
            device_id=left, device_id_type=pl.DeviceIdType.MESH,
        )
        a1.start()
        b1.start()

        a1.wait_recv()
        a2 = pltpu.make_async_remote_copy(
            src_ref=ra1, dst_ref=ra2,
            send_sem=rs_sems.at[2], recv_sem=rr_sems.at[2],
            device_id=right, device_id_type=pl.DeviceIdType.MESH,
        )
        a2.start()
        out_ref[pl.ds(plft * R, R), :] = ra1[...].astype(jnp.float32)

        b1.wait_recv()
        out_ref[pl.ds(pr * R, R), :] = rb1[...].astype(jnp.float32)

        a2.wait_recv()
        out_ref[pl.ds(((p + 2) % NXY) * R, R), :] = ra2[...].astype(
            jnp.float32)

        a1.wait_send()
        b1.wait_send()
        a2.wait_send()

    out = pl.pallas_call(
        body,
        out_shape=jax.ShapeDtypeStruct((S, D), jnp.float32),
        in_specs=[pl.BlockSpec(memory_space=pltpu.VMEM)] * 8,
        out_specs=pl.BlockSpec(memory_space=pltpu.VMEM),
        scratch_shapes=[
            pltpu.VMEM((S, Dc), bf),
            pltpu.VMEM((S, Dc), bf),
            pltpu.VMEM((Dc, D), bf),
            pltpu.VMEM((Dc, D), bf),
            pltpu.VMEM((R, D), bf),
            pltpu.VMEM((R, D), bf),
            pltpu.VMEM((R, D), bf),
            pltpu.VMEM((R, D), bf),
            pltpu.SemaphoreType.DMA((3,)),
            pltpu.SemaphoreType.DMA((3,)),
            pltpu.SemaphoreType.DMA((3,)),
            pltpu.SemaphoreType.DMA((3,)),
        ],
        compiler_params=pltpu.CompilerParams(
            collective_id=0, vmem_limit_bytes=100 * 2**20
        ),
    )(x2, wdkv, wuk, wuv, wq, wqr, wkr, wo)

    return out.reshape(B, S, D)
